# baseline (speedup 1.0000x reference)
import functools

import jax
import jax.numpy as jnp
from jax import lax
from jax.experimental import pallas as pl
from jax.experimental.pallas import tpu as pltpu
from jax.experimental.pallas import tpu_sc as plsc

_T, _B, _D = 50, 1024, 1000
_NW = 32
_NDT = 4
_UNITS = _T * _NDT


@functools.partial(
    pl.kernel,
    out_type=jax.ShapeDtypeStruct((_T, 125, 8, 8, 128), jnp.float32),
    mesh=plsc.VectorSubcoreMesh(core_axis_name="c", subcore_axis_name="s"),
    compiler_params=pltpu.CompilerParams(use_tc_tiling_on_sc=False, needs_layout_passes=False),
    scratch_types=[
        pltpu.VMEM((32 * _D,), jnp.float32),
        pltpu.VMEM((_T * _B,), jnp.int32),
        pltpu.VMEM((2, 8, 8, 128), jnp.float32),
        pltpu.SemaphoreType.DMA,
        pltpu.SemaphoreType.DMA,
    ],
)
def _sc_bigram(tabt_hbm, idxt_hbm, out_hbm, subtab, idxv, buf, ws0, ws1):
    wid = lax.axis_index("s") * 2 + lax.axis_index("c")

    pltpu.sync_copy(tabt_hbm.at[pl.ds(wid * 32 * _D, 32 * _D)], subtab)
    pltpu.sync_copy(idxt_hbm, idxv)

    wsem = (ws0, ws1)

    def unit(u, b):
        dl = (_NDT - 1) - u // _T
        t = u % _T
        dt = jnp.minimum(wid * _NDT + dl, 124)
        for bt in range(8):
            for ch in range(8):
                iv = idxv[pl.ds(t * _B + bt * 128 + ch * 16, 16)]
                for dp in range(8):
                    vals = plsc.load_gather(subtab, [iv + (dl * 8 + dp) * _D])
                    buf[b, bt, dp, pl.ds(ch * 16, 16)] = vals
        pltpu.async_copy(buf.at[b], out_hbm.at[t, dt], wsem[b])

    def wwait(b):
        pltpu.make_async_copy(buf.at[b], out_hbm.at[0, 0], wsem[b]).wait()

    def body(i, carry):
        @pl.when(i > 0)
        def _():
            wwait(0)
        unit(2 * i, 0)

        @pl.when(i > 0)
        def _():
            wwait(1)
        unit(2 * i + 1, 1)
        return carry

    lax.fori_loop(0, _UNITS // 2, body, 0)
    wwait(0)
    wwait(1)


def kernel(idx, token_embedding_table):
    tabt = jnp.pad(token_embedding_table.T, ((0, 24), (0, 0))).reshape(-1)
    idxt = idx.T.reshape(-1)
    out_lin = _sc_bigram(tabt, idxt)
    return out_lin.transpose(2, 4, 0, 1, 3).reshape(_B, _T, _D)


# speedup vs baseline: 1.0516x; 1.0516x over previous
import functools

import jax
import jax.numpy as jnp
from jax import lax
from jax.experimental import pallas as pl
from jax.experimental.pallas import tpu as pltpu
from jax.experimental.pallas import tpu_sc as plsc

_D = 1000
_NW = 32
_B, _T = 1024, 50
_BPW = _B // _NW


@functools.partial(
    pl.kernel,
    out_type=jax.ShapeDtypeStruct((_B, _T, _D), jnp.float32),
    mesh=plsc.VectorSubcoreMesh(core_axis_name="c", subcore_axis_name="s"),
    compiler_params=pltpu.CompilerParams(use_tc_tiling_on_sc=False),
    scratch_types=[
        pltpu.VMEM((_BPW * 56,), jnp.int32),
        pltpu.VMEM((2, _T, _D), jnp.float32),
        pltpu.SemaphoreType.DMA,
        pltpu.SemaphoreType.DMA,
        pltpu.SemaphoreType.DMA,
        pltpu.SemaphoreType.DMA,
    ],
)
def _sc_gather(table_hbm, idx_hbm, out_hbm, idx_v, buf, g0, g1, w0, w1):
    sid = lax.axis_index("s")
    wid = sid * 2 + lax.axis_index("c")

    pltpu.sync_copy(idx_hbm.at[pl.ds(wid * _BPW * 56, _BPW * 56)], idx_v)

    gsem = (g0, g1)
    wsem = (w0, w1)

    def start_g(j, s):
        pltpu.async_copy(
            table_hbm.at[idx_v.at[pl.ds(j * 56, _T)]], buf.at[s], gsem[s]
        )

    def wait_g(s):
        pltpu.make_async_copy(
            table_hbm.at[idx_v.at[pl.ds(0, _T)]], buf.at[s], gsem[s]
        ).wait()

    def start_w(j, s):
        pltpu.async_copy(buf.at[s], out_hbm.at[wid * _BPW + j], wsem[s])

    def wait_w(s):
        pltpu.make_async_copy(buf.at[s], out_hbm.at[0], wsem[s]).wait()

    start_g(0, 0)
    start_g(1, 1)

    def body(i, carry):
        j0 = 2 * i
        wait_g(0)
        start_w(j0, 0)
        wait_g(1)
        start_w(j0 + 1, 1)
        wait_w(0)
        start_g(jnp.minimum(j0 + 2, _BPW - 1), 0)
        wait_w(1)
        start_g(jnp.minimum(j0 + 3, _BPW - 1), 1)
        return carry

    lax.fori_loop(0, _BPW // 2, body, 0)
    wait_g(0)
    wait_g(1)


def kernel(idx, token_embedding_table):
    idx_p = jnp.pad(idx, ((0, 0), (0, 6))).reshape(-1)
    return _sc_gather(token_embedding_table, idx_p)


# speedup vs baseline: 1.1093x; 1.0548x over previous
import functools

import jax
import jax.numpy as jnp
from jax import lax
from jax.experimental import pallas as pl
from jax.experimental.pallas import tpu as pltpu
from jax.experimental.pallas import tpu_sc as plsc

_D = 1000
_NW = 32
_B, _T = 1024, 50
_CH = 25
_NC = _B * _T // _CH
_CPW = _NC // _NW


@functools.partial(
    pl.kernel,
    out_type=jax.ShapeDtypeStruct((_NC, _CH, _D), jnp.float32),
    mesh=plsc.VectorSubcoreMesh(core_axis_name="c", subcore_axis_name="s"),
    compiler_params=pltpu.CompilerParams(use_tc_tiling_on_sc=False),
    scratch_types=[
        pltpu.VMEM((_CPW * 32,), jnp.int32),
        pltpu.VMEM((2, _CH, _D), jnp.float32),
        pltpu.VMEM_SHARED((1000, _D), jnp.float32),
        pltpu.SemaphoreType.DMA,
        pltpu.SemaphoreType.DMA,
        pltpu.SemaphoreType.DMA,
        pltpu.SemaphoreType.DMA,
    ],
)
def _sc_gather(table_hbm, idx_hbm, out_hbm, idx_v, buf, tab_sp, g0, g1, w0, w1):
    sid = lax.axis_index("s")
    wid = sid * 2 + lax.axis_index("c")

    @pl.when(sid == 0)
    def _():
        pltpu.sync_copy(table_hbm, tab_sp)

    pltpu.sync_copy(idx_hbm.at[pl.ds(wid * _CPW * 32, _CPW * 32)], idx_v)
    plsc.subcore_barrier()

    gsem = (g0, g1)
    wsem = (w0, w1)

    def start_g(j, s):
        pltpu.async_copy(
            tab_sp.at[idx_v.at[pl.ds(j * 32, _CH)]], buf.at[s], gsem[s]
        )

    def wait_g(s):
        pltpu.make_async_copy(
            tab_sp.at[idx_v.at[pl.ds(0, _CH)]], buf.at[s], gsem[s]
        ).wait()

    def start_w(j, s):
        pltpu.async_copy(buf.at[s], out_hbm.at[wid * _CPW + j], wsem[s])

    def wait_w(s):
        pltpu.make_async_copy(buf.at[s], out_hbm.at[0], wsem[s]).wait()

    start_g(0, 0)
    start_g(1, 1)

    def body(i, carry):
        j0 = 2 * i
        wait_g(0)
        start_w(j0, 0)
        wait_g(1)
        start_w(j0 + 1, 1)
        wait_w(0)
        start_g(jnp.minimum(j0 + 2, _CPW - 1), 0)
        wait_w(1)
        start_g(jnp.minimum(j0 + 3, _CPW - 1), 1)
        return carry

    lax.fori_loop(0, _CPW // 2, body, 0)
    wait_g(0)
    wait_g(1)


def kernel(idx, token_embedding_table):
    idx_p = jnp.pad(idx.reshape(_B, 2, _CH), ((0, 0), (0, 0), (0, 7))).reshape(-1)
    out = _sc_gather(token_embedding_table, idx_p)
    return out.reshape(_B, _T, _D)


# speedup vs baseline: 1.1128x; 1.0032x over previous
import functools

import jax
import jax.numpy as jnp
from jax import lax
from jax.experimental import pallas as pl
from jax.experimental.pallas import tpu as pltpu
from jax.experimental.pallas import tpu_sc as plsc

_D = 1000
_N = 51200
_NW = 32
_RPW = _N // _NW
_CHUNK = 32
_NCHUNK = _RPW // _CHUNK


_B, _T = 1024, 50
_BPW = _B // _NW


@functools.partial(
    pl.kernel,
    out_type=jax.ShapeDtypeStruct((_B, _T, _D), jnp.float32),
    mesh=plsc.VectorSubcoreMesh(core_axis_name="c", subcore_axis_name="s"),
    compiler_params=pltpu.CompilerParams(use_tc_tiling_on_sc=False),
    scratch_types=[
        pltpu.VMEM((_BPW * 56,), jnp.int32),
        pltpu.VMEM((_T, _D), jnp.float32),
        pltpu.VMEM_SHARED((1000, _D), jnp.float32),
        pltpu.SemaphoreType.DMA,
        pltpu.SemaphoreType.DMA,
    ],
)
def _sc_gather(table_hbm, idx_hbm, out_hbm, idx_v, buf, tab_sp, gsem, wsem):
    sid = lax.axis_index("s")
    wid = sid * 2 + lax.axis_index("c")

    @pl.when(sid == 0)
    def _():
        pltpu.sync_copy(table_hbm, tab_sp)

    pltpu.sync_copy(idx_hbm.at[pl.ds(wid * _BPW * 56, _BPW * 56)], idx_v)
    plsc.subcore_barrier()

    def body(j, carry):
        pltpu.async_copy(
            tab_sp.at[idx_v.at[pl.ds(j * 56, _T)]], buf, gsem
        ).wait()
        pltpu.async_copy(buf, out_hbm.at[wid * _BPW + j], wsem).wait()
        return carry

    lax.fori_loop(0, _BPW, body, 0)


def kernel(idx, token_embedding_table):
    idx_p = jnp.pad(idx, ((0, 0), (0, 6))).reshape(-1)
    return _sc_gather(token_embedding_table, idx_p)
